# baseline (speedup 1.0000x reference)
import functools

import jax
import jax.numpy as jnp
from jax import lax
from jax.experimental import pallas as pl
from jax.experimental.pallas import tpu as pltpu
from jax.experimental.pallas import tpu_sc as plsc

PAD_ID = 0
START_ID = 0
LANES = 16
SUB = 16


@functools.partial(jax.jit, static_argnames=("B", "S", "D"))
def _embed_lookup(ids2d, wte, wpe, B, S, D):
    NC, NS = 2, 16
    NW = NC * NS
    CH = S // NW
    nst = CH // SUB
    G = B * SUB
    MCH = B * S // NW

    mesh = plsc.VectorSubcoreMesh(core_axis_name="c", subcore_axis_name="s")

    @functools.partial(
        pl.kernel,
        mesh=mesh,
        out_type=(
            jax.ShapeDtypeStruct((B, S, D), jnp.float32),
            jax.ShapeDtypeStruct((B, S), jnp.int32),
            jax.ShapeDtypeStruct((B * S,), jnp.float32),
        ),
        scratch_types=[
            pltpu.VMEM((B, CH), jnp.int32),
            pltpu.VMEM((MCH,), jnp.float32),
            pltpu.VMEM((G,), jnp.int32),
            pltpu.VMEM((G,), jnp.int32),
            pltpu.VMEM((G, D), jnp.float32),
            pltpu.VMEM((G, D), jnp.float32),
            pltpu.VMEM((SUB, D), jnp.float32),
            pltpu.VMEM((SUB, D), jnp.float32),
            pltpu.SemaphoreType.DMA,
            pltpu.SemaphoreType.DMA,
            pltpu.SemaphoreType.DMA,
            pltpu.SemaphoreType.DMA,
            pltpu.SemaphoreType.DMA,
            pltpu.SemaphoreType.DMA,
            pltpu.SemaphoreType.DMA,
        ],
    )
    def k(ids_hbm, wte_hbm, wpe_hbm, out_hbm, idsout_hbm, mask_hbm,
          idx_v, zbuf, l0, l1, r0, r1, w0, w1,
          g0, g1, p0, p1, s0_, s1_, msem):
        lists, rows, wpeb = [l0, l1], [r0, r1], [w0, w1]
        gsem, psem, wsem = [g0, g1], [p0, p1], [s0_, s1_]
        wid = lax.axis_index("s") * NC + lax.axis_index("c")
        s0 = wid * CH

        idx_copies = [
            pltpu.async_copy(ids_hbm.at[b, pl.ds(s0, CH)], idx_v.at[b], msem)
            for b in range(B)
        ]
        for c in idx_copies:
            c.wait()

        gathers = [None, None]
        wloads = [None, None]
        writes = [[], []]

        def issue(h):
            p = h % 2
            for wcopy in writes[p]:
                wcopy.wait()
            writes[p] = []
            for b in range(B):
                lists[p][pl.ds(b * SUB, SUB)] = idx_v[b, pl.ds(h * SUB, SUB)]
            gathers[p] = pltpu.async_copy(wte_hbm.at[lists[p]], rows[p], gsem[p])
            wloads[p] = pltpu.async_copy(
                wpe_hbm.at[pl.ds(s0 + h * SUB, SUB), :], wpeb[p], psem[p]
            )

        def run_add(p):
            @plsc.parallel_loop(0, SUB // 2, unroll=1)
            def _(i):
                for u in range(2):
                    iu = i + u * (SUB // 2)
                    for jj in range(D // LANES):
                        sl = pl.ds(jj * LANES, LANES)
                        w = wpeb[p][iu, sl]
                        for b in range(B):
                            plsc.addupdate(rows[p].at[b * SUB + iu, sl], w)

        issue(0)

        zv = jnp.zeros((LANES,), jnp.float32)
        for q in range(MCH // LANES):
            zbuf[pl.ds(q * LANES, LANES)] = zv
        mwrite = pltpu.async_copy(
            zbuf, mask_hbm.at[pl.ds(wid * MCH, MCH)], msem
        )

        for h in range(nst):
            p = h % 2
            if h + 1 < nst:
                issue(h + 1)
            gathers[p].wait()
            wloads[p].wait()
            run_add(p)
            writes[p] = [
                pltpu.async_copy(
                    rows[p].at[pl.ds(b * SUB, SUB), :],
                    out_hbm.at[b, pl.ds(s0 + h * SUB, SUB), :],
                    wsem[p],
                )
                for b in range(B)
            ]

        tails = [
            pltpu.async_copy(idx_v.at[b], idsout_hbm.at[b, pl.ds(s0, CH)], msem)
            for b in range(B)
        ]
        for p in range(2):
            for wcopy in writes[p]:
                wcopy.wait()
        for t in tails:
            t.wait()
        mwrite.wait()

    return k(ids2d, wte, wpe)


def kernel(encoder_hidden_states, labels, metadata, wte, wpe):
    B, S = labels.shape
    D = wte.shape[1]

    ids = jnp.concatenate(
        [jnp.full((B, 1), START_ID, labels.dtype), labels[:, :-1]], axis=1
    )
    ids = jnp.where(ids == -100, PAD_ID, ids)

    token_emb, ids_out, mask_flat = _embed_lookup(ids, wte, wpe, B, S, D)

    enc_b, enc_s, _ = encoder_hidden_states.shape
    encoder_extended_attention_mask = mask_flat.reshape(enc_b, 1, 1, enc_s)

    return (
        encoder_hidden_states,
        token_emb,
        encoder_extended_attention_mask,
        metadata,
        ids_out,
        labels,
    )


# speedup vs baseline: 1.0597x; 1.0597x over previous
import functools

import jax
import jax.numpy as jnp
from jax import lax
from jax.experimental import pallas as pl
from jax.experimental.pallas import tpu as pltpu
from jax.experimental.pallas import tpu_sc as plsc

PAD_ID = 0
START_ID = 0
LANES = 16
SUB = 16


@functools.partial(jax.jit, static_argnames=("B", "S", "D"))
def _embed_lookup(ids2d, wte, wpe, B, S, D):
    NC, NS = 2, 16
    NW = NC * NS
    CH = S // NW
    nst = CH // SUB
    G = B * SUB
    MCH = B * S // NW

    mesh = plsc.VectorSubcoreMesh(core_axis_name="c", subcore_axis_name="s")

    @functools.partial(
        pl.kernel,
        mesh=mesh,
        out_type=(
            jax.ShapeDtypeStruct((B, S, D), jnp.float32),
            jax.ShapeDtypeStruct((B, S), jnp.int32),
            jax.ShapeDtypeStruct((B * S,), jnp.float32),
        ),
        scratch_types=[
            pltpu.VMEM((B, CH), jnp.int32),
            pltpu.VMEM((MCH,), jnp.float32),
            pltpu.VMEM((G,), jnp.int32),
            pltpu.VMEM((G,), jnp.int32),
            pltpu.VMEM((G, D), jnp.float32),
            pltpu.VMEM((G, D), jnp.float32),
            pltpu.VMEM((SUB, D), jnp.float32),
            pltpu.VMEM((SUB, D), jnp.float32),
            pltpu.SemaphoreType.DMA,
            pltpu.SemaphoreType.DMA,
            pltpu.SemaphoreType.DMA,
            pltpu.SemaphoreType.DMA,
            pltpu.SemaphoreType.DMA,
            pltpu.SemaphoreType.DMA,
            pltpu.SemaphoreType.DMA,
        ],
    )
    def k(ids_hbm, wte_hbm, wpe_hbm, out_hbm, idsout_hbm, mask_hbm,
          idx_v, zbuf, l0, l1, r0, r1, w0, w1,
          g0, g1, p0, p1, s0_, s1_, msem):
        lists, rows, wpeb = [l0, l1], [r0, r1], [w0, w1]
        gsem, psem, wsem = [g0, g1], [p0, p1], [s0_, s1_]
        wid = lax.axis_index("s") * NC + lax.axis_index("c")
        s0 = wid * CH

        idx_copies = [
            pltpu.async_copy(ids_hbm.at[b, pl.ds(s0, CH)], idx_v.at[b], msem)
            for b in range(B)
        ]
        for c in idx_copies:
            c.wait()

        gathers = [None, None]
        wloads = [None, None]
        writes = [[], []]

        def issue(h):
            p = h % 2
            for wcopy in writes[p]:
                wcopy.wait()
            writes[p] = []
            for b in range(B):
                lists[p][pl.ds(b * SUB, SUB)] = idx_v[b, pl.ds(h * SUB, SUB)]
            gathers[p] = pltpu.async_copy(wte_hbm.at[lists[p]], rows[p], gsem[p])
            wloads[p] = pltpu.async_copy(
                wpe_hbm.at[pl.ds(s0 + h * SUB, SUB), :], wpeb[p], psem[p]
            )

        def run_add(p):
            @plsc.parallel_loop(0, SUB, unroll=1)
            def _(i):
                for jj in range(D // LANES):
                    sl = pl.ds(jj * LANES, LANES)
                    w = wpeb[p][i, sl]
                    for b in range(B):
                        plsc.addupdate(rows[p].at[b * SUB + i, sl], w)

        issue(0)

        zv = jnp.zeros((LANES,), jnp.float32)
        for q in range(MCH // LANES):
            zbuf[pl.ds(q * LANES, LANES)] = zv
        mwrite = pltpu.async_copy(
            zbuf, mask_hbm.at[pl.ds(wid * MCH, MCH)], msem
        )

        for h in range(nst):
            p = h % 2
            if h + 1 < nst:
                issue(h + 1)
            gathers[p].wait()
            wloads[p].wait()
            run_add(p)
            writes[p] = [
                pltpu.async_copy(
                    rows[p].at[pl.ds(b * SUB, SUB), :],
                    out_hbm.at[b, pl.ds(s0 + h * SUB, SUB), :],
                    wsem[p],
                )
                for b in range(B)
            ]

        tails = [
            pltpu.async_copy(idx_v.at[b], idsout_hbm.at[b, pl.ds(s0, CH)], msem)
            for b in range(B)
        ]
        for p in range(2):
            for wcopy in writes[p]:
                wcopy.wait()
        for t in tails:
            t.wait()
        mwrite.wait()

    return k(ids2d, wte, wpe)


def kernel(encoder_hidden_states, labels, metadata, wte, wpe):
    B, S = labels.shape
    D = wte.shape[1]

    ids = jnp.concatenate(
        [jnp.full((B, 1), START_ID, labels.dtype), labels[:, :-1]], axis=1
    )
    ids = jnp.where(ids == -100, PAD_ID, ids)

    token_emb, ids_out, mask_flat = _embed_lookup(ids, wte, wpe, B, S, D)

    enc_b, enc_s, _ = encoder_hidden_states.shape
    encoder_extended_attention_mask = mask_flat.reshape(enc_b, 1, 1, enc_s)

    return (
        encoder_hidden_states,
        token_emb,
        encoder_extended_attention_mask,
        metadata,
        ids_out,
        labels,
    )
